# Initial kernel scaffold; baseline (speedup 1.0000x reference)
import jax
import jax.numpy as jnp
from jax.experimental import pallas as pl


def kernel(x, W_enc, fano_W, fano_b, line_weights, W_to8, codebook, W_from8, W_dec):
    raise NotImplementedError("write your pallas kernel here")


# speedup vs baseline: 1.5783x; 1.5783x over previous
import jax
import jax.numpy as jnp
from jax.experimental import pallas as pl
from jax.experimental.pallas import tpu as pltpu

_FANO_LINES = [(0, 1, 2), (0, 3, 4), (0, 5, 6), (1, 3, 5), (1, 4, 6), (2, 3, 6), (2, 4, 5)]
_LINES_OF = [[0, 1, 2], [0, 3, 4], [0, 5, 6], [1, 3, 5], [1, 4, 6], [2, 3, 6], [2, 4, 5]]
_NUM_LAYERS = 3
_VQ_LEVELS = 8
_CDIM = 8
_TDIM = 7 * _CDIM
_K = 240

_HI = jax.lax.Precision.HIGHEST


def _main_body(x_ref, Wenc_ref, BD_ref, bcat_ref, wrep_ref, Wto8_ref,
               cb_ref, cbt_ref, Wfrom8_ref, Wdec_ref, out_ref):
    f32 = jnp.float32
    B = x_ref.shape[0]
    h = jnp.dot(x_ref[...], Wenc_ref[...], preferred_element_type=f32)
    for l in range(_NUM_LAYERS):
        cols = []
        for (i, j, k) in _FANO_LINES:
            cols.append(h[:, i * _CDIM:(i + 1) * _CDIM])
            cols.append(h[:, j * _CDIM:(j + 1) * _CDIM])
            cols.append(h[:, k * _CDIM:(k + 1) * _CDIM])
        triple_all = jnp.concatenate(cols, axis=1)
        outs = jnp.dot(triple_all, BD_ref[l], preferred_element_type=f32)
        outs = (outs + bcat_ref[l:l + 1, :]) * wrep_ref[l:l + 1, :]
        res = []
        for c in range(7):
            a, b, g = _LINES_OF[c]
            res.append(outs[:, a * _CDIM:(a + 1) * _CDIM]
                       + outs[:, b * _CDIM:(b + 1) * _CDIM]
                       + outs[:, g * _CDIM:(g + 1) * _CDIM])
        h = jnp.concatenate(res, axis=1) / 3.0 + h
    z = jnp.dot(h, Wto8_ref[...], preferred_element_type=f32)

    C = cb_ref[...]
    Ct = cbt_ref[...]
    cn = jnp.sum(Ct * Ct, axis=0, keepdims=True)
    iota = jax.lax.broadcasted_iota(jnp.int32, (B, _K), 1).astype(f32)
    r = z
    q = jnp.zeros_like(z)
    for _ in range(_VQ_LEVELS):
        s = jnp.dot(r, Ct, preferred_element_type=f32, precision=_HI)
        d = cn - 2.0 * s
        m = jnp.min(d, axis=1, keepdims=True)
        idx = jnp.min(jnp.where(d == m, iota, float(_K)), axis=1,
                      keepdims=True)
        oh = (iota == idx).astype(f32)
        cq = jnp.dot(oh, C, preferred_element_type=f32, precision=_HI)
        q = q + cq
        r = r - cq
    zq = z + (q - z)
    t = jnp.dot(zq, Wfrom8_ref[...], preferred_element_type=f32)
    out_ref[...] = jnp.dot(t, Wdec_ref[...], preferred_element_type=f32)


def kernel(x, W_enc, fano_W, fano_b, line_weights, W_to8, codebook, W_from8, W_dec):
    n, bulk = x.shape
    f32 = jnp.float32

    ws = jnp.stack([jax.nn.softmax(line_weights[l]) for l in range(_NUM_LAYERS)])
    BD = jnp.zeros((_NUM_LAYERS, 7 * 3 * _CDIM, _TDIM), f32)
    for li in range(7):
        BD = BD.at[:, 24 * li:24 * li + 24, _CDIM * li:_CDIM * (li + 1)].set(
            fano_W[:, li])
    b_cat = fano_b.reshape(_NUM_LAYERS, _TDIM)
    w_rep = jnp.repeat(ws, _CDIM, axis=1)

    blk = 2048
    grid = (n // blk,)
    out = pl.pallas_call(
        _main_body,
        grid=grid,
        in_specs=[
            pl.BlockSpec((blk, bulk), lambda i: (i, 0)),
            pl.BlockSpec((bulk, _TDIM), lambda i: (0, 0)),
            pl.BlockSpec((_NUM_LAYERS, 168, _TDIM), lambda i: (0, 0, 0)),
            pl.BlockSpec((_NUM_LAYERS, _TDIM), lambda i: (0, 0)),
            pl.BlockSpec((_NUM_LAYERS, _TDIM), lambda i: (0, 0)),
            pl.BlockSpec((_TDIM, _CDIM), lambda i: (0, 0)),
            pl.BlockSpec((_K, _CDIM), lambda i: (0, 0)),
            pl.BlockSpec((_CDIM, _K), lambda i: (0, 0)),
            pl.BlockSpec((_CDIM, _TDIM), lambda i: (0, 0)),
            pl.BlockSpec((_TDIM, bulk), lambda i: (0, 0)),
        ],
        out_specs=pl.BlockSpec((blk, bulk), lambda i: (i, 0)),
        out_shape=jax.ShapeDtypeStruct((n, bulk), f32),
        compiler_params=pltpu.CompilerParams(
            dimension_semantics=("arbitrary",),
        ),
    )(x, W_enc, BD, b_cat, w_rep, W_to8, codebook, codebook.T, W_from8, W_dec)
    return out


# speedup vs baseline: 2.0215x; 1.2808x over previous
import jax
import jax.numpy as jnp
from jax.experimental import pallas as pl
from jax.experimental.pallas import tpu as pltpu

_FANO_LINES = [(0, 1, 2), (0, 3, 4), (0, 5, 6), (1, 3, 5), (1, 4, 6), (2, 3, 6), (2, 4, 5)]
_LINES_OF = [[0, 1, 2], [0, 3, 4], [0, 5, 6], [1, 3, 5], [1, 4, 6], [2, 3, 6], [2, 4, 5]]
_NUM_LAYERS = 3
_VQ_LEVELS = 8
_CDIM = 8
_TDIM = 7 * _CDIM
_K = 240

_HI = jax.lax.Precision.HIGHEST


def _main_body(x_ref, Wenc_ref, M_ref, bcat_ref, wrep_ref, Wto8_ref,
               cbt_ref, c1_ref, c2_ref, c3_ref, Wfrom8_ref, Wdec_ref,
               out_ref):
    f32 = jnp.float32
    B = x_ref.shape[0]
    h = jnp.dot(x_ref[...], Wenc_ref[...], preferred_element_type=f32)
    for l in range(_NUM_LAYERS):
        outs = jnp.dot(h, M_ref[l], preferred_element_type=f32)
        outs = (outs + bcat_ref[l:l + 1, :]) * wrep_ref[l:l + 1, :]
        res = []
        for c in range(7):
            a, b, g = _LINES_OF[c]
            res.append(outs[:, a * _CDIM:(a + 1) * _CDIM]
                       + outs[:, b * _CDIM:(b + 1) * _CDIM]
                       + outs[:, g * _CDIM:(g + 1) * _CDIM])
        h = jnp.concatenate(res, axis=1) / 3.0 + h
    z = jnp.dot(h, Wto8_ref[...], preferred_element_type=f32)

    Ct = cbt_ref[...]
    cn = jnp.sum(Ct * Ct, axis=0, keepdims=True)
    iota = jax.lax.broadcasted_iota(jnp.int32, (B, _K), 1).astype(f32)
    r = z
    q = jnp.zeros_like(z)
    for _ in range(_VQ_LEVELS):
        s = jnp.dot(r, Ct, preferred_element_type=f32, precision=_HI)
        d = cn - 2.0 * s
        m = jnp.min(d, axis=1, keepdims=True)
        idx = jnp.min(jnp.where(d == m, iota, float(_K)), axis=1,
                      keepdims=True)
        oh = (iota == idx).astype(f32)
        cq = (jnp.dot(oh, c1_ref[...], preferred_element_type=f32)
              + jnp.dot(oh, c2_ref[...], preferred_element_type=f32)
              + jnp.dot(oh, c3_ref[...], preferred_element_type=f32))
        q = q + cq
        r = r - cq
    zq = z + (q - z)
    t = jnp.dot(zq, Wfrom8_ref[...], preferred_element_type=f32)
    out_ref[...] = jnp.dot(t, Wdec_ref[...], preferred_element_type=f32)


def kernel(x, W_enc, fano_W, fano_b, line_weights, W_to8, codebook, W_from8, W_dec):
    n, bulk = x.shape
    f32 = jnp.float32

    ws = jnp.stack([jax.nn.softmax(line_weights[l]) for l in range(_NUM_LAYERS)])
    cols = []
    for li, (i, j, k) in enumerate(_FANO_LINES):
        blk = jnp.zeros((_NUM_LAYERS, _TDIM, _CDIM), f32)
        blk = blk.at[:, _CDIM * i:_CDIM * (i + 1), :].set(fano_W[:, li, 0:8])
        blk = blk.at[:, _CDIM * j:_CDIM * (j + 1), :].set(fano_W[:, li, 8:16])
        blk = blk.at[:, _CDIM * k:_CDIM * (k + 1), :].set(fano_W[:, li, 16:24])
        cols.append(blk)
    M = jnp.concatenate(cols, axis=2)
    b_cat = fano_b.reshape(_NUM_LAYERS, _TDIM)
    w_rep = jnp.repeat(ws, _CDIM, axis=1)

    mask = jnp.uint32(0xFFFF0000)
    bits = jax.lax.bitcast_convert_type(codebook, jnp.uint32)
    c1 = jax.lax.bitcast_convert_type(bits & mask, f32)
    r1 = codebook - c1
    c2 = jax.lax.bitcast_convert_type(
        jax.lax.bitcast_convert_type(r1, jnp.uint32) & mask, f32)
    c3 = r1 - c2

    blk = 2048
    grid = (n // blk,)
    out = pl.pallas_call(
        _main_body,
        grid=grid,
        in_specs=[
            pl.BlockSpec((blk, bulk), lambda i: (i, 0)),
            pl.BlockSpec((bulk, _TDIM), lambda i: (0, 0)),
            pl.BlockSpec((_NUM_LAYERS, _TDIM, _TDIM), lambda i: (0, 0, 0)),
            pl.BlockSpec((_NUM_LAYERS, _TDIM), lambda i: (0, 0)),
            pl.BlockSpec((_NUM_LAYERS, _TDIM), lambda i: (0, 0)),
            pl.BlockSpec((_TDIM, _CDIM), lambda i: (0, 0)),
            pl.BlockSpec((_CDIM, _K), lambda i: (0, 0)),
            pl.BlockSpec((_K, _CDIM), lambda i: (0, 0)),
            pl.BlockSpec((_K, _CDIM), lambda i: (0, 0)),
            pl.BlockSpec((_K, _CDIM), lambda i: (0, 0)),
            pl.BlockSpec((_CDIM, _TDIM), lambda i: (0, 0)),
            pl.BlockSpec((_TDIM, bulk), lambda i: (0, 0)),
        ],
        out_specs=pl.BlockSpec((blk, bulk), lambda i: (i, 0)),
        out_shape=jax.ShapeDtypeStruct((n, bulk), f32),
        compiler_params=pltpu.CompilerParams(
            dimension_semantics=("arbitrary",),
        ),
    )(x, W_enc, M, b_cat, w_rep, W_to8, codebook.T, c1, c2, c3, W_from8, W_dec)
    return out


# speedup vs baseline: 4.2081x; 2.0817x over previous
import jax
import jax.numpy as jnp
from jax.experimental import pallas as pl
from jax.experimental.pallas import tpu as pltpu

_FANO_LINES = [(0, 1, 2), (0, 3, 4), (0, 5, 6), (1, 3, 5), (1, 4, 6), (2, 3, 6), (2, 4, 5)]
_LINES_OF = [[0, 1, 2], [0, 3, 4], [0, 5, 6], [1, 3, 5], [1, 4, 6], [2, 3, 6], [2, 4, 5]]
_NUM_LAYERS = 3
_VQ_LEVELS = 8
_CDIM = 8
_TDIM = 7 * _CDIM
_K = 240

_HI = jax.lax.Precision.HIGHEST


def _main_body(x_ref, Wenc_ref, M_ref, bcat_ref, wrep_ref, Wto8_ref,
               cb_ref, c1t_ref, c2t_ref, c3t_ref, Wfrom8_ref, Wdec_ref,
               out_ref):
    f32 = jnp.float32
    B = x_ref.shape[0]
    h = jnp.dot(x_ref[...], Wenc_ref[...], preferred_element_type=f32)
    for l in range(_NUM_LAYERS):
        outs = jnp.dot(h, M_ref[l], preferred_element_type=f32)
        outs = (outs + bcat_ref[l:l + 1, :]) * wrep_ref[l:l + 1, :]
        res = []
        for c in range(7):
            a, b, g = _LINES_OF[c]
            res.append(outs[:, a * _CDIM:(a + 1) * _CDIM]
                       + outs[:, b * _CDIM:(b + 1) * _CDIM]
                       + outs[:, g * _CDIM:(g + 1) * _CDIM])
        h = jnp.concatenate(res, axis=1) / 3.0 + h
    z = jnp.dot(h, Wto8_ref[...], preferred_element_type=f32)

    C = cb_ref[...]
    cn = jnp.sum(C * C, axis=1, keepdims=True)
    zT = z.T
    iota = jax.lax.broadcasted_iota(jnp.int32, (_K, B), 0).astype(f32)
    rT = zT
    qT = jnp.zeros_like(zT)
    for _ in range(_VQ_LEVELS):
        sT = jnp.dot(C, rT, preferred_element_type=f32, precision=_HI)
        d = cn - 2.0 * sT
        m = jnp.min(d, axis=0, keepdims=True)
        idx = jnp.min(jnp.where(d == m, iota, float(_K)), axis=0,
                      keepdims=True)
        oh = (iota == idx).astype(f32)
        cqT = (jnp.dot(c1t_ref[...], oh, preferred_element_type=f32)
               + jnp.dot(c2t_ref[...], oh, preferred_element_type=f32)
               + jnp.dot(c3t_ref[...], oh, preferred_element_type=f32))
        qT = qT + cqT
        rT = rT - cqT
    zqT = zT + (qT - zT)
    zq = zqT.T
    t = jnp.dot(zq, Wfrom8_ref[...], preferred_element_type=f32)
    out_ref[...] = jnp.dot(t, Wdec_ref[...], preferred_element_type=f32)


def kernel(x, W_enc, fano_W, fano_b, line_weights, W_to8, codebook, W_from8, W_dec):
    n, bulk = x.shape
    f32 = jnp.float32

    ws = jnp.stack([jax.nn.softmax(line_weights[l]) for l in range(_NUM_LAYERS)])
    cols = []
    for li, (i, j, k) in enumerate(_FANO_LINES):
        blk = jnp.zeros((_NUM_LAYERS, _TDIM, _CDIM), f32)
        blk = blk.at[:, _CDIM * i:_CDIM * (i + 1), :].set(fano_W[:, li, 0:8])
        blk = blk.at[:, _CDIM * j:_CDIM * (j + 1), :].set(fano_W[:, li, 8:16])
        blk = blk.at[:, _CDIM * k:_CDIM * (k + 1), :].set(fano_W[:, li, 16:24])
        cols.append(blk)
    M = jnp.concatenate(cols, axis=2)
    b_cat = fano_b.reshape(_NUM_LAYERS, _TDIM)
    w_rep = jnp.repeat(ws, _CDIM, axis=1)

    mask = jnp.uint32(0xFFFF0000)
    bits = jax.lax.bitcast_convert_type(codebook, jnp.uint32)
    c1 = jax.lax.bitcast_convert_type(bits & mask, f32)
    r1 = codebook - c1
    c2 = jax.lax.bitcast_convert_type(
        jax.lax.bitcast_convert_type(r1, jnp.uint32) & mask, f32)
    c3 = r1 - c2

    blk = 2048
    grid = (n // blk,)
    out = pl.pallas_call(
        _main_body,
        grid=grid,
        in_specs=[
            pl.BlockSpec((blk, bulk), lambda i: (i, 0)),
            pl.BlockSpec((bulk, _TDIM), lambda i: (0, 0)),
            pl.BlockSpec((_NUM_LAYERS, _TDIM, _TDIM), lambda i: (0, 0, 0)),
            pl.BlockSpec((_NUM_LAYERS, _TDIM), lambda i: (0, 0)),
            pl.BlockSpec((_NUM_LAYERS, _TDIM), lambda i: (0, 0)),
            pl.BlockSpec((_TDIM, _CDIM), lambda i: (0, 0)),
            pl.BlockSpec((_K, _CDIM), lambda i: (0, 0)),
            pl.BlockSpec((_CDIM, _K), lambda i: (0, 0)),
            pl.BlockSpec((_CDIM, _K), lambda i: (0, 0)),
            pl.BlockSpec((_CDIM, _K), lambda i: (0, 0)),
            pl.BlockSpec((_CDIM, _TDIM), lambda i: (0, 0)),
            pl.BlockSpec((_TDIM, bulk), lambda i: (0, 0)),
        ],
        out_specs=pl.BlockSpec((blk, bulk), lambda i: (i, 0)),
        out_shape=jax.ShapeDtypeStruct((n, bulk), f32),
        compiler_params=pltpu.CompilerParams(
            dimension_semantics=("arbitrary",),
        ),
    )(x, W_enc, M, b_cat, w_rep, W_to8, codebook, c1.T, c2.T, c3.T, W_from8, W_dec)
    return out
